# Initial kernel scaffold; baseline (speedup 1.0000x reference)
import jax
import jax.numpy as jnp
from jax.experimental import pallas as pl


def kernel(x, y, params):
    raise NotImplementedError("write your pallas kernel here")


# speedup vs baseline: 6.6590x; 6.6590x over previous
import functools

import jax
import jax.numpy as jnp
import numpy as np
from jax import lax
from jax.experimental import pallas as pl
from jax.experimental.pallas import tpu as pltpu
from jax.experimental.pallas import tpu_sc as plsc

_K = 16
_SLOPE = 0.2
_NBLK = 4
_NEG = -3.0e38


def _knn_body(xf_ref, xx_ref, out_ref, *, tn, n, k):
    b = pl.program_id(0)
    t = pl.program_id(1)
    xall = xf_ref[0]
    xt = xf_ref[0, pl.ds(t * tn, tn), :]
    inner = lax.dot_general(xt, xall, (((1,), (1,)), ((), ())),
                            preferred_element_type=jnp.float32)
    xx = xx_ref[0, 0, :]
    xxt = xx_ref[0, 0, pl.ds(t * tn, tn)]
    dist = 2.0 * inner - xxt[:, None] - xx[None, :]
    iota = lax.broadcasted_iota(jnp.int32, (tn, n), 1)
    cols = []
    d = dist
    for _ in range(k):
        m = jnp.max(d, axis=1, keepdims=True)
        am = jnp.min(jnp.where(d >= m, iota, n), axis=1)
        cols.append(am[:, None])
        d = jnp.where(iota == am[:, None], _NEG, d)
    idx = jnp.concatenate(cols, axis=1)
    out_ref[0] = idx + b * n


def _knn(F):
    B, N, C3 = F.shape
    C = C3 // 3
    xfc = jnp.transpose(F.reshape(B, N, 3, C), (0, 3, 2, 1)).reshape(B, C3, N)
    xx = jnp.sum(xfc * xfc, axis=1).reshape(B, 1, N)
    xn = jnp.transpose(xfc, (0, 2, 1))
    tn = min(256, N)
    return pl.pallas_call(
        functools.partial(_knn_body, tn=tn, n=N, k=_K),
        grid=(B, N // tn),
        in_specs=[pl.BlockSpec((1, N, C3), lambda b, t: (b, 0, 0)),
                  pl.BlockSpec((1, 1, N), lambda b, t: (b, 0, 0))],
        out_specs=pl.BlockSpec((1, tn, _K), lambda b, t: (b, t, 0)),
        out_shape=jax.ShapeDtypeStruct((B, N, _K), jnp.int32),
    )(xn, xx)


@functools.lru_cache(maxsize=None)
def _gather_call(R, D, M):
    info = plsc.get_sparse_core_info()
    nw = info.num_cores * info.num_subcores
    bpw = M // nw
    chunk = min(bpw, max(8, 2 ** int(np.log2(98304 // D))))
    nch = bpw // chunk
    mesh = plsc.VectorSubcoreMesh(core_axis_name="c", subcore_axis_name="s")

    @functools.partial(
        pl.kernel, mesh=mesh,
        out_type=jax.ShapeDtypeStruct((M, D), jnp.float32),
        scratch_types=[
            pltpu.VMEM((chunk,), jnp.int32),
            pltpu.VMEM((chunk, D), jnp.float32),
            pltpu.SemaphoreType.DMA,
        ],
    )
    def gk(table_hbm, idx_hbm, out_hbm, idx_v, rows_v, sem):
        wid = lax.axis_index("s") * info.num_cores + lax.axis_index("c")
        base = wid * bpw
        for i in range(nch):
            off = base + i * chunk
            pltpu.sync_copy(idx_hbm.at[pl.ds(off, chunk)], idx_v)
            pltpu.async_copy(table_hbm.at[idx_v], rows_v, sem).wait()
            pltpu.sync_copy(rows_v, out_hbm.at[pl.ds(off, chunk)])

    return gk


def _gather_rows(table, gidx_flat):
    R, D = table.shape
    Dp = -(-D // 128) * 128
    if Dp != D:
        table = jnp.pad(table, ((0, 0), (0, Dp - D)))
    M = gidx_flat.shape[0]
    return _gather_call(R, Dp, M)(table, gidx_flat)


def _edge_body(g_ref, f_ref, w_ref, o_ref, *, ci, co, k, tn):
    G = g_ref[...]
    F = f_ref[0]
    Wt = w_ref[...]
    ps, qs = [], []
    dot, d2 = None, None
    for d in range(3):
        fd = G[:, d * ci:(d + 1) * ci].reshape(tn, k, ci)
        cd = F[:, d * ci:(d + 1) * ci][:, None, :]
        e = jnp.concatenate([fd - cd, jnp.broadcast_to(cd, fd.shape)], axis=2)
        e2 = e.reshape(tn * k, 2 * ci).astype(jnp.bfloat16)
        pq = jnp.dot(e2, Wt, preferred_element_type=jnp.float32)
        p, q = pq[:, :co], pq[:, co:]
        ps.append(p)
        qs.append(q)
        dot = p * q if dot is None else dot + p * q
        d2 = q * q if d2 is None else d2 + q * q
    inv = dot / (d2 + 1e-9)
    for d in range(3):
        p, q = ps[d], qs[d]
        od = _SLOPE * p + (1.0 - _SLOPE) * jnp.where(dot >= 0, p, p - inv * q)
        od3 = od.reshape(tn, k, co)
        acc = od3[:, 0, :]
        for j in range(1, k):
            acc = acc + od3[:, j, :]
        o_ref[0, :, d * co:(d + 1) * co] = acc / float(k)


def _edge(G, F, Wt, ci, co):
    B, N, D0 = F.shape
    Dg = G.shape[-1]
    tn = min(128, N)
    return pl.pallas_call(
        functools.partial(_edge_body, ci=ci, co=co, k=_K, tn=tn),
        grid=(B, N // tn),
        in_specs=[
            pl.BlockSpec((tn * _K, Dg), lambda b, t, nt=N // tn: (b * nt + t, 0)),
            pl.BlockSpec((1, tn, D0), lambda b, t: (b, t, 0)),
            pl.BlockSpec(Wt.shape, lambda b, t: (0, 0)),
        ],
        out_specs=pl.BlockSpec((1, tn, 3 * co), lambda b, t: (b, t, 0)),
        out_shape=jax.ShapeDtypeStruct((B, N, 3 * co), jnp.float32),
    )(G, F, Wt)


def _leaky_from(ps, qs):
    dot = ps[0] * qs[0] + ps[1] * qs[1] + ps[2] * qs[2]
    d2 = qs[0] * qs[0] + qs[1] * qs[1] + qs[2] * qs[2]
    inv = dot / (d2 + 1e-9)
    return [_SLOPE * p + (1.0 - _SLOPE) * jnp.where(dot >= 0, p, p - inv * q)
            for p, q in zip(ps, qs)]


def _chnorm3(xs):
    n2 = jnp.sqrt(xs[0] * xs[0] + xs[1] * xs[1] + xs[2] * xs[2])
    n1 = jnp.sqrt(jnp.sum(n2 * n2, axis=-1, keepdims=True))
    return [(x / (n2 + 1e-12)) * (n2 / (n1 + 1e-12)) for x in xs]


def _cross_body(fx_ref, fy_ref, g_ref, wq_ref, wkv_ref, o_ref, *, c, k, tn):
    Fx = fx_ref[0]
    Fy = fy_ref[0]
    G = g_ref[...]
    Wq = wq_ref[...]
    Wkv = wkv_ref[...]
    qp, qq = [], []
    for d in range(3):
        xd = Fx[:, d * c:(d + 1) * c].astype(jnp.bfloat16)
        pq = jnp.dot(xd, Wq, preferred_element_type=jnp.float32)
        qp.append(pq[:, :c])
        qq.append(pq[:, c:])
    Q = _chnorm3(_leaky_from(qp, qq))
    kp, kq, vp, vq = [], [], [], []
    for d in range(3):
        fd = G[:, d * c:(d + 1) * c].reshape(tn, k, c)
        cd = Fy[:, d * c:(d + 1) * c][:, None, :]
        e = jnp.concatenate([fd - cd, jnp.broadcast_to(cd, fd.shape)], axis=2)
        e2 = e.reshape(tn * k, 2 * c).astype(jnp.bfloat16)
        pq = jnp.dot(e2, Wkv, preferred_element_type=jnp.float32)
        kp.append(pq[:, 0 * c:1 * c])
        kq.append(pq[:, 1 * c:2 * c])
        vp.append(pq[:, 2 * c:3 * c])
        vq.append(pq[:, 3 * c:4 * c])
    Ky = _chnorm3(_leaky_from(kp, kq))
    Vy = _leaky_from(vp, vq)
    qk = None
    for d in range(3):
        part = Ky[d].reshape(tn, k, c) * Q[d][:, None, :]
        qk = part if qk is None else qk + part
    z = qk / np.sqrt(3.0 * c)
    m = jnp.max(z, axis=1, keepdims=True)
    e = jnp.exp(z - m)
    s = e[:, 0:1, :]
    for j in range(1, k):
        s = s + e[:, j:j + 1, :]
    att = e / s
    for d in range(3):
        av = att * Vy[d].reshape(tn, k, c)
        od = av[:, 0, :]
        for j in range(1, k):
            od = od + av[:, j, :]
        o_ref[0, :, d * c:(d + 1) * c] = Fx[:, d * c:(d + 1) * c] + od


def _cross(Fx, Fy, G, Wq, Wkv):
    B, N, C3 = Fx.shape
    c = C3 // 3
    Dg = G.shape[-1]
    tn = min(128, N)
    return pl.pallas_call(
        functools.partial(_cross_body, c=c, k=_K, tn=tn),
        grid=(B, N // tn),
        in_specs=[
            pl.BlockSpec((1, tn, C3), lambda b, t: (b, t, 0)),
            pl.BlockSpec((1, tn, C3), lambda b, t: (b, t, 0)),
            pl.BlockSpec((tn * _K, Dg), lambda b, t, nt=N // tn: (b * nt + t, 0)),
            pl.BlockSpec(Wq.shape, lambda b, t: (0, 0)),
            pl.BlockSpec(Wkv.shape, lambda b, t: (0, 0)),
        ],
        out_specs=pl.BlockSpec((1, tn, C3), lambda b, t: (b, t, 0)),
        out_shape=jax.ShapeDtypeStruct((B, N, C3), jnp.float32),
    )(Fx, Fy, G, Wq, Wkv)


def _global_body(f_ref, w_ref, o_ref, *, c, co):
    F = f_ref[0]
    Wt = w_ref[...]
    mean = jnp.mean(F, axis=0, keepdims=True)
    ps, qs = [], []
    for d in range(3):
        fd = F[:, d * c:(d + 1) * c]
        md = jnp.broadcast_to(mean[:, d * c:(d + 1) * c], fd.shape)
        e = jnp.concatenate([fd, md], axis=1).astype(jnp.bfloat16)
        pq = jnp.dot(e, Wt, preferred_element_type=jnp.float32)
        ps.append(pq[:, :co])
        qs.append(pq[:, co:])
    out = _leaky_from(ps, qs)
    for d in range(3):
        o_ref[0, :, d * co:(d + 1) * co] = out[d]


def _global(F, Wt, co):
    B, N, C3 = F.shape
    return pl.pallas_call(
        functools.partial(_global_body, c=C3 // 3, co=co),
        grid=(B,),
        in_specs=[
            pl.BlockSpec((1, N, C3), lambda b: (b, 0, 0)),
            pl.BlockSpec(Wt.shape, lambda b: (0, 0)),
        ],
        out_specs=pl.BlockSpec((1, N, 3 * co), lambda b: (b, 0, 0)),
        out_shape=jax.ShapeDtypeStruct((B, N, 3 * co), jnp.float32),
    )(F, Wt)


def _inv_scores(FxN, FyN):
    B, N, C3 = FxN.shape
    C = C3 // 3
    fx = jnp.transpose(FxN.reshape(B, N, 3, C), (0, 3, 2, 1))
    fy = jnp.transpose(FyN.reshape(B, N, 3, C), (0, 3, 2, 1))
    fx_mean = jnp.mean(fx, axis=1)
    fy_mean = jnp.mean(fy, axis=1)
    fx_par = fx_mean / (jnp.linalg.norm(fx_mean, axis=1, keepdims=True) + 1e-6)
    fy_par = fy_mean / (jnp.linalg.norm(fy_mean, axis=1, keepdims=True) + 1e-6)
    phi_x = jnp.einsum('bcdn,bdn->bnc', fx, fx_par)
    phi_y = jnp.einsum('bcdn,bdn->bnc', fy, fy_par)
    return jnp.einsum('bnc,bnc->bn', phi_x, phi_y)


def _vn_leaky4(x, W, U):
    p = jnp.einsum('oc,bcd->bod', W, x)
    d = jnp.einsum('oc,bcd->bod', U, x)
    dot = jnp.sum(p * d, axis=2, keepdims=True)
    d2 = jnp.sum(d * d, axis=2, keepdims=True)
    mask = (dot >= 0).astype(p.dtype)
    p_neg = p - (dot / (d2 + 1e-9)) * d
    return _SLOPE * p + (1.0 - _SLOPE) * (mask * p + (1.0 - mask) * p_neg)


def _pad16(F):
    B, N, D = F.shape
    if D % 16 == 0:
        return F
    return jnp.pad(F, ((0, 0), (0, 0), (0, 16 - D % 16)))


def kernel(x, y, params):
    B, _, N = x.shape
    bf = jnp.bfloat16
    Fx = jnp.transpose(x, (0, 2, 1))
    Fy = jnp.transpose(y, (0, 2, 1))
    ci = 1
    mxs, mys = [], []
    for i in range(_NBLK):
        p = params['block%d' % i]
        co = p['dW'].shape[0]
        C3 = 3 * co
        Wd = jnp.concatenate([p['dW'].T, p['dU'].T], axis=1).astype(bf)
        idx_x = _knn(Fx).reshape(-1)
        idx_y = _knn(Fy).reshape(-1)
        Fxp = _pad16(Fx)
        Fyp = _pad16(Fy)
        D0 = Fxp.shape[-1]
        Gx = _gather_rows(Fxp.reshape(B * N, D0), idx_x)
        Gy = _gather_rows(Fyp.reshape(B * N, D0), idx_y)
        Fx1 = _edge(Gx, Fxp, Wd, ci, co)
        Fy1 = _edge(Gy, Fyp, Wd, ci, co)
        Wq = jnp.concatenate([p['qW'].T, p['qU'].T], axis=1).astype(bf)
        Wkv = jnp.concatenate([p['kW'].T, p['kU'].T,
                               p['vW'].T, p['vU'].T], axis=1).astype(bf)
        idx_yg = _knn(Fy1).reshape(-1)
        Gyg = _gather_rows(Fy1.reshape(B * N, C3), idx_yg)
        Fx2 = _cross(Fx1, Fy1, Gyg, Wq, Wkv)
        idx_xg = _knn(Fx2).reshape(-1)
        Gxg = _gather_rows(Fx2.reshape(B * N, C3), idx_xg)
        Fy2 = _cross(Fy1, Fx2, Gxg, Wq, Wkv)
        Wg = jnp.concatenate([p['gW'].T, p['gU'].T], axis=1).astype(bf)
        Fx3 = _global(Fx2, Wg, co)
        Fy3 = _global(Fy2, Wg, co)
        z = _inv_scores(Fx3, Fy3)
        _, idx = lax.top_k(z, N // 2)
        gidx = (idx + (jnp.arange(B, dtype=idx.dtype) * N)[:, None]).reshape(-1)
        tab = jnp.concatenate([Fx3, Fy3], axis=-1).reshape(B * N, 2 * C3)
        g = _gather_rows(tab, gidx)
        g = g.reshape(B, N // 2, -1)
        Fx, Fy = g[..., :C3], g[..., C3:2 * C3]
        N //= 2
        ci = co
        mx = jnp.mean(Fx, axis=1).reshape(B, 3, co)
        my = jnp.mean(Fy, axis=1).reshape(B, 3, co)
        mxs.append(jnp.transpose(mx, (0, 2, 1)))
        mys.append(jnp.transpose(my, (0, 2, 1)))
    catx = jnp.concatenate(mxs, axis=1)
    caty = jnp.concatenate(mys, axis=1)
    hx = _vn_leaky4(catx, params['hW'], params['hU'])
    hy = _vn_leaky4(caty, params['hW'], params['hU'])
    H = jnp.einsum('bcd,bce->bde', hx, hy)
    u, _, vh = jnp.linalg.svd(H, full_matrices=False)
    R = jnp.matmul(u, vh)
    S = jnp.linalg.norm(hy, axis=1) / jnp.linalg.norm(hx, axis=1)
    return R, S
